# Initial kernel scaffold; baseline (speedup 1.0000x reference)
import jax
import jax.numpy as jnp
from jax.experimental import pallas as pl


def kernel(features, idx):
    raise NotImplementedError("write your pallas kernel here")


# speedup vs baseline: 754.9537x; 754.9537x over previous
import functools

import jax
import jax.numpy as jnp
from jax import lax
from jax.experimental import pallas as pl
from jax.experimental.pallas import tpu as pltpu
from jax.experimental.pallas import tpu_sc as plsc

NC = 2
NS = 16
NW = NC * NS
LANES = 16

CH = 2048
GROUP = 4


def _sc_gather(features, idx2):
    B, C, N = features.shape
    _, J = idx2.shape
    n_chunk = J // CH
    cpt = C // (NW // B)
    n_group = cpt // GROUP
    tiles_per_b = NW // B

    def body(feat_hbm, idx_hbm, out_hbm, idx_v, rows_v, obuf_v):
        cid = lax.axis_index("c")
        sid = lax.axis_index("s")
        wid = sid * NC + cid
        b = wid // tiles_per_b
        cq = wid % tiles_per_b

        for g in range(n_group):
            c_base = cq * cpt + g * GROUP
            for r in range(GROUP):
                pltpu.sync_copy(feat_hbm.at[b, c_base + r], rows_v[r])

            def chunk_body(ch, _):
                pltpu.sync_copy(idx_hbm.at[b, pl.ds(ch * CH, CH)], idx_v)

                def vbody(v, _):
                    iv = idx_v[pl.ds(v * LANES, LANES)]
                    for r in range(GROUP):
                        obuf_v[r][pl.ds(v * LANES, LANES)] = plsc.load_gather(
                            rows_v[r], [iv])
                    return _

                lax.fori_loop(0, CH // LANES, vbody, 0, unroll=2)
                for r in range(GROUP):
                    pltpu.sync_copy(obuf_v[r],
                                    out_hbm.at[b, c_base + r, pl.ds(ch * CH, CH)])
                return _

            lax.fori_loop(0, n_chunk, chunk_body, 0)

    mesh = plsc.VectorSubcoreMesh(core_axis_name="c", subcore_axis_name="s",
                                  num_cores=NC, num_subcores=NS)
    call = pl.kernel(
        body,
        out_type=jax.ShapeDtypeStruct((B, C, J), jnp.float32),
        mesh=mesh,
        compiler_params=pltpu.CompilerParams(needs_layout_passes=False),
        scratch_types=[
            pltpu.VMEM((CH,), jnp.int32),
            [pltpu.VMEM((N,), jnp.float32) for _ in range(GROUP)],
            [pltpu.VMEM((CH,), jnp.float32) for _ in range(GROUP)],
        ],
    )
    return call(features, idx2)


def kernel(features, idx):
    B, C, N = features.shape
    _, npoint, nsample = idx.shape
    J = npoint * nsample
    idx2 = idx.reshape(B, J).astype(jnp.int32)
    out = _sc_gather(features, idx2)
    return out.reshape(B, C, npoint, nsample)


# speedup vs baseline: 1562.0599x; 2.0691x over previous
import jax
import jax.numpy as jnp
from jax import lax
from jax.experimental import pallas as pl
from jax.experimental.pallas import tpu as pltpu
from jax.experimental.pallas import tpu_sc as plsc

NC = 2
NS = 16
NW = NC * NS
LANES = 16

CH = 2048
GROUP = 4


def _sc_gather(features, idx2):
    B, C, N = features.shape
    _, J = idx2.shape
    n_chunk = J // CH
    tiles_per_b = NW // B
    cpt = C // tiles_per_b
    n_group = cpt // GROUP

    def body(feat_hbm, idx_hbm, out_hbm, idx_v, rows_v, obuf_v,
             sem_row, sem_idx, sem_out):
        cid = lax.axis_index("c")
        sid = lax.axis_index("s")
        wid = sid * NC + cid
        b = wid // tiles_per_b
        cq = wid % tiles_per_b

        def wait_out(p):
            for r in range(GROUP):
                pltpu.make_async_copy(
                    obuf_v[p][r], out_hbm.at[b, 0, pl.ds(0, CH)],
                    sem_out[p]).wait()

        for g in range(n_group):
            c_base = cq * cpt + g * GROUP
            for r in range(GROUP):
                pltpu.async_copy(feat_hbm.at[b, c_base + r], rows_v[r],
                                 sem_row)
            if g > 0:
                wait_out(0)
                wait_out(1)
            for r in range(GROUP):
                pltpu.make_async_copy(feat_hbm.at[b, c_base + r], rows_v[r],
                                      sem_row).wait()
            pltpu.async_copy(idx_hbm.at[b, pl.ds(0, CH)], idx_v[0], sem_idx)

            def half_body(half, _):
                for p in (0, 1):
                    ch = half * 2 + p

                    @pl.when(ch + 1 < n_chunk)
                    def _start_next_idx():
                        pltpu.async_copy(
                            idx_hbm.at[b, pl.ds((ch + 1) * CH, CH)],
                            idx_v[1 - p], sem_idx)

                    pltpu.make_async_copy(idx_hbm.at[b, pl.ds(0, CH)],
                                          idx_v[p], sem_idx).wait()

                    @pl.when(half > 0)
                    def _drain_prev():
                        wait_out(p)

                    @plsc.parallel_loop(0, CH // LANES, 1, unroll=4)
                    def _gather(v):
                        iv = idx_v[p][pl.ds(v * LANES, LANES)]
                        for r in range(GROUP):
                            obuf_v[p][r][pl.ds(v * LANES, LANES)] = (
                                plsc.load_gather(rows_v[r], [iv]))

                    for r in range(GROUP):
                        pltpu.async_copy(
                            obuf_v[p][r],
                            out_hbm.at[b, c_base + r, pl.ds(ch * CH, CH)],
                            sem_out[p])
                return _

            lax.fori_loop(0, n_chunk // 2, half_body, 0)

        wait_out(0)
        wait_out(1)

    mesh = plsc.VectorSubcoreMesh(core_axis_name="c", subcore_axis_name="s",
                                  num_cores=NC, num_subcores=NS)
    call = pl.kernel(
        body,
        out_type=jax.ShapeDtypeStruct((B, C, J), jnp.float32),
        mesh=mesh,
        compiler_params=pltpu.CompilerParams(needs_layout_passes=False),
        scratch_types=[
            [pltpu.VMEM((CH,), jnp.int32) for _ in range(2)],
            [pltpu.VMEM((N,), jnp.float32) for _ in range(GROUP)],
            [[pltpu.VMEM((CH,), jnp.float32) for _ in range(GROUP)]
             for _ in range(2)],
            pltpu.SemaphoreType.DMA,
            pltpu.SemaphoreType.DMA,
            [pltpu.SemaphoreType.DMA for _ in range(2)],
        ],
    )
    return call(features, idx2)


def kernel(features, idx):
    B, C, N = features.shape
    _, npoint, nsample = idx.shape
    J = npoint * nsample
    idx2 = idx.reshape(B, J).astype(jnp.int32)
    out = _sc_gather(features, idx2)
    return out.reshape(B, C, npoint, nsample)
